# baseline (device time: 267765 ns/iter reference)
import functools

import jax
import jax.numpy as jnp
from jax import lax
from jax.experimental import pallas as pl
from jax.experimental.pallas import tpu as pltpu

N_DEV = 4


def _flash_partial_body(q_ref, k_ref, v_ref, o_ref, m_ref, l_ref, *, scale, d):
    h = pl.program_id(1)
    q = q_ref[0].astype(jnp.bfloat16)
    k = k_ref[0].astype(jnp.bfloat16)
    v = v_ref[0].astype(jnp.bfloat16)

    s = lax.dot_general(
        q, k, (((1,), (1,)), ((), ())), preferred_element_type=jnp.float32
    )
    s = s * scale
    m = jnp.max(s, axis=-1, keepdims=True)
    p = jnp.exp(s - m)
    l = jnp.sum(p, axis=-1, keepdims=True)
    o = lax.dot_general(
        p.astype(jnp.bfloat16), v, (((1,), (0,)), ((), ())),
        preferred_element_type=jnp.float32,
    )

    o_ref[0, :, pl.ds(h * d, d)] = o.astype(jnp.bfloat16)
    sq, nh = m_ref.shape[1], m_ref.shape[2]
    lane = lax.broadcasted_iota(jnp.int32, (sq, nh), 1)
    m_ref[0] = jnp.where(lane == h, jnp.broadcast_to(m, (sq, nh)), m_ref[0])
    l_ref[0] = jnp.where(lane == h, jnp.broadcast_to(l, (sq, nh)), l_ref[0])


def _flash_partial(Q, K, V):
    B, Sq, H, D = Q.shape
    Skv = K.shape[1]
    scale = D**-0.5
    Q2 = Q.reshape(B, Sq, H * D)
    K2 = K.reshape(B, Skv, H * D)
    V2 = V.reshape(B, Skv, H * D)
    body = functools.partial(_flash_partial_body, scale=scale, d=D)
    o, m, l = pl.pallas_call(
        body,
        grid=(B, H),
        in_specs=[
            pl.BlockSpec((1, Sq, D), lambda b, h: (b, 0, h)),
            pl.BlockSpec((1, Skv, D), lambda b, h: (b, 0, h)),
            pl.BlockSpec((1, Skv, D), lambda b, h: (b, 0, h)),
        ],
        out_specs=[
            pl.BlockSpec((1, Sq, H * D), lambda b, h: (b, 0, 0)),
            pl.BlockSpec((1, Sq, H), lambda b, h: (b, 0, 0)),
            pl.BlockSpec((1, Sq, H), lambda b, h: (b, 0, 0)),
        ],
        out_shape=[
            jax.ShapeDtypeStruct((B, Sq, H * D), jnp.bfloat16),
            jax.ShapeDtypeStruct((B, Sq, H), jnp.float32),
            jax.ShapeDtypeStruct((B, Sq, H), jnp.float32),
        ],
        compiler_params=pltpu.CompilerParams(
            dimension_semantics=("arbitrary", "arbitrary"),
        ),
    )(Q2, K2, V2)
    return o.reshape(B, Sq, H, D), m, l


def _allreduce_body(
    o_ref, m_ref, l_ref, out_ref, obuf, mbuf, lbuf, co, cm, cl, send_sems, recv_sems
):
    my = lax.axis_index("i")
    p1 = jnp.bitwise_xor(my, 1)
    p2 = 3 - my

    barrier = pltpu.get_barrier_semaphore()
    for nbr in (p1, p2):
        pl.semaphore_signal(
            barrier, inc=1, device_id=(nbr,), device_id_type=pl.DeviceIdType.MESH
        )
    pl.semaphore_wait(barrier, 2)

    def exchange(r, partner, src_o, src_m, src_l):
        copies = []
        for j, (src, dst) in enumerate(
            ((src_o, obuf), (src_m, mbuf), (src_l, lbuf))
        ):
            rdma = pltpu.make_async_remote_copy(
                src_ref=src,
                dst_ref=dst.at[r],
                send_sem=send_sems.at[r, j],
                recv_sem=recv_sems.at[r, j],
                device_id=(partner,),
                device_id_type=pl.DeviceIdType.MESH,
            )
            rdma.start()
            copies.append(rdma)
        for rdma in copies:
            rdma.wait()

    def combine(r, src_o, src_m, src_l):
        m_a = src_m[...]
        m_b = mbuf[r]
        m_new = jnp.maximum(m_a, m_b)
        w_a = jnp.exp(m_a - m_new)
        w_b = jnp.exp(m_b - m_new)
        l_new = src_l[...] * w_a + lbuf[r] * w_b
        o_new = (
            src_o[...].astype(jnp.float32) * w_a[..., None]
            + obuf[r].astype(jnp.float32) * w_b[..., None]
        )
        return o_new, m_new, l_new

    exchange(0, p1, o_ref, m_ref, l_ref)
    o1, m1, l1 = combine(0, o_ref, m_ref, l_ref)
    co[...] = o1.astype(jnp.bfloat16)
    cm[...] = m1
    cl[...] = l1
    exchange(1, p2, co, cm, cl)
    o2, _, l2 = combine(1, co, cm, cl)
    out_ref[...] = o2 / l2[..., None]


def _combine_allreduce(o, m, l):
    B, Sq, H, D = o.shape
    return pl.pallas_call(
        _allreduce_body,
        out_shape=jax.ShapeDtypeStruct((B, Sq, H, D), jnp.float32),
        in_specs=[pl.BlockSpec(memory_space=pltpu.VMEM)] * 3,
        out_specs=pl.BlockSpec(memory_space=pltpu.VMEM),
        scratch_shapes=[
            pltpu.VMEM((2, B, Sq, H, D), jnp.bfloat16),
            pltpu.VMEM((2, B, Sq, H), jnp.float32),
            pltpu.VMEM((2, B, Sq, H), jnp.float32),
            pltpu.VMEM((B, Sq, H, D), jnp.bfloat16),
            pltpu.VMEM((B, Sq, H), jnp.float32),
            pltpu.VMEM((B, Sq, H), jnp.float32),
            pltpu.SemaphoreType.DMA((2, 3)),
            pltpu.SemaphoreType.DMA((2, 3)),
        ],
        compiler_params=pltpu.CompilerParams(collective_id=0),
    )(o, m, l)


def kernel(Q, K, V):
    o, m, l = _flash_partial(Q, K, V)
    return _combine_allreduce(o, m, l)


# device time: 59503 ns/iter; 4.5000x vs baseline; 4.5000x over previous
import functools

import jax
import jax.numpy as jnp
from jax import lax
from jax.experimental import pallas as pl
from jax.experimental.pallas import tpu as pltpu

N_DEV = 4
KV_CHUNK = 512
NBUF = 2


def _flash_partial_body(
    q_hbm, k_hbm, v_hbm, o_ref, m_ref, l_ref, qb, kb, vb, qsem, sems, *, scale
):
    B, Sq, H, D = q_hbm.shape
    Skv = k_hbm.shape[1]
    C = KV_CHUNK
    NC = Skv // C

    pltpu.make_async_copy(q_hbm, qb, qsem).start()

    def start(i):
        b, c = divmod(i, NC)
        slot = i % NBUF
        for h in range(H):
            pltpu.make_async_copy(
                k_hbm.at[b, pl.ds(c * C, C), h, :], kb.at[slot, h], sems.at[slot, 0]
            ).start()
            pltpu.make_async_copy(
                v_hbm.at[b, pl.ds(c * C, C), h, :], vb.at[slot, h], sems.at[slot, 1]
            ).start()

    NIT = B * NC
    for s in range(min(NBUF, NIT)):
        start(s)

    pltpu.make_async_copy(q_hbm, qb, qsem).wait()

    for b in range(B):
        qs = (jnp.transpose(qb[b], (1, 0, 2)) * scale).astype(jnp.bfloat16)
        acc = None
        m = None
        l = None
        for c in range(NC):
            i = b * NC + c
            slot = i % NBUF
            for h in range(H):
                pltpu.make_async_copy(
                    k_hbm.at[0, pl.ds(0, C), 0, :], kb.at[0, 0], sems.at[slot, 0]
                ).wait()
                pltpu.make_async_copy(
                    v_hbm.at[0, pl.ds(0, C), 0, :], vb.at[0, 0], sems.at[slot, 1]
                ).wait()
            k = kb[slot].astype(jnp.bfloat16)
            v = vb[slot].astype(jnp.bfloat16)
            nxt = i + NBUF
            if nxt < NIT:
                start(nxt)
            s = lax.dot_general(
                qs, k, (((2,), (2,)), ((0,), (0,))),
                preferred_element_type=jnp.float32,
            )
            m_c = jnp.max(s, axis=-1)
            if acc is None:
                m = m_c
                p = jnp.exp(s - m[:, :, None])
                l = jnp.sum(p, axis=-1)
                acc = lax.dot_general(
                    p.astype(jnp.bfloat16), v, (((2,), (1,)), ((0,), (0,))),
                    preferred_element_type=jnp.float32,
                )
            else:
                m_new = jnp.maximum(m, m_c)
                alpha = jnp.exp(m - m_new)
                p = jnp.exp(s - m_new[:, :, None])
                pv = lax.dot_general(
                    p.astype(jnp.bfloat16), v, (((2,), (1,)), ((0,), (0,))),
                    preferred_element_type=jnp.float32,
                )
                acc = acc * alpha[:, :, None] + pv
                l = l * alpha + jnp.sum(p, axis=-1)
                m = m_new
        o_ref[b] = jnp.transpose(acc, (1, 0, 2)).astype(jnp.bfloat16)
        m_ref[b] = m.T
        l_ref[b] = l.T


def _flash_partial(Q, K, V):
    B, Sq, H, D = Q.shape
    Skv = K.shape[1]
    scale = D**-0.5
    body = functools.partial(_flash_partial_body, scale=scale)
    return pl.pallas_call(
        body,
        in_specs=[
            pl.BlockSpec(memory_space=pl.ANY),
            pl.BlockSpec(memory_space=pl.ANY),
            pl.BlockSpec(memory_space=pl.ANY),
        ],
        out_specs=[
            pl.BlockSpec(memory_space=pltpu.VMEM),
            pl.BlockSpec(memory_space=pltpu.VMEM),
            pl.BlockSpec(memory_space=pltpu.VMEM),
        ],
        out_shape=[
            jax.ShapeDtypeStruct((B, Sq, H, D), jnp.bfloat16),
            jax.ShapeDtypeStruct((B, Sq, H), jnp.float32),
            jax.ShapeDtypeStruct((B, Sq, H), jnp.float32),
        ],
        scratch_shapes=[
            pltpu.VMEM((B, Sq, H, D), jnp.float32),
            pltpu.VMEM((NBUF, H, KV_CHUNK, D), jnp.float32),
            pltpu.VMEM((NBUF, H, KV_CHUNK, D), jnp.float32),
            pltpu.SemaphoreType.DMA,
            pltpu.SemaphoreType.DMA((NBUF, 2)),
        ],
    )(Q, K, V)


def _allreduce_body(
    o_ref, m_ref, l_ref, out_ref, obuf, mbuf, lbuf, co, cm, cl, send_sems, recv_sems
):
    my = lax.axis_index("i")
    p1 = jnp.bitwise_xor(my, 1)
    p2 = 3 - my

    barrier = pltpu.get_barrier_semaphore()
    for nbr in (p1, p2):
        pl.semaphore_signal(
            barrier, inc=1, device_id=(nbr,), device_id_type=pl.DeviceIdType.MESH
        )
    pl.semaphore_wait(barrier, 2)

    def exchange(r, partner, src_o, src_m, src_l):
        copies = []
        for j, (src, dst) in enumerate(
            ((src_o, obuf), (src_m, mbuf), (src_l, lbuf))
        ):
            rdma = pltpu.make_async_remote_copy(
                src_ref=src,
                dst_ref=dst.at[r],
                send_sem=send_sems.at[r, j],
                recv_sem=recv_sems.at[r, j],
                device_id=(partner,),
                device_id_type=pl.DeviceIdType.MESH,
            )
            rdma.start()
            copies.append(rdma)
        for rdma in copies:
            rdma.wait()

    def combine(r, src_o, src_m, src_l):
        m_a = src_m[...]
        m_b = mbuf[r]
        m_new = jnp.maximum(m_a, m_b)
        w_a = jnp.exp(m_a - m_new)
        w_b = jnp.exp(m_b - m_new)
        l_new = src_l[...] * w_a + lbuf[r] * w_b
        o_new = (
            src_o[...].astype(jnp.float32) * w_a[..., None]
            + obuf[r].astype(jnp.float32) * w_b[..., None]
        )
        return o_new, m_new, l_new

    exchange(0, p1, o_ref, m_ref, l_ref)
    o1, m1, l1 = combine(0, o_ref, m_ref, l_ref)
    co[...] = o1.astype(jnp.bfloat16)
    cm[...] = m1
    cl[...] = l1
    exchange(1, p2, co, cm, cl)
    o2, _, l2 = combine(1, co, cm, cl)
    out_ref[...] = o2 / l2[..., None]


def _combine_allreduce(o, m, l):
    B, Sq, H, D = o.shape
    return pl.pallas_call(
        _allreduce_body,
        out_shape=jax.ShapeDtypeStruct((B, Sq, H, D), jnp.float32),
        in_specs=[pl.BlockSpec(memory_space=pltpu.VMEM)] * 3,
        out_specs=pl.BlockSpec(memory_space=pltpu.VMEM),
        scratch_shapes=[
            pltpu.VMEM((2, B, Sq, H, D), jnp.bfloat16),
            pltpu.VMEM((2, B, Sq, H), jnp.float32),
            pltpu.VMEM((2, B, Sq, H), jnp.float32),
            pltpu.VMEM((B, Sq, H, D), jnp.bfloat16),
            pltpu.VMEM((B, Sq, H), jnp.float32),
            pltpu.VMEM((B, Sq, H), jnp.float32),
            pltpu.SemaphoreType.DMA((2, 3)),
            pltpu.SemaphoreType.DMA((2, 3)),
        ],
        compiler_params=pltpu.CompilerParams(collective_id=0),
    )(o, m, l)


def kernel(Q, K, V):
    o, m, l = _flash_partial(Q, K, V)
    return _combine_allreduce(o, m, l)


# device time: 50733 ns/iter; 5.2779x vs baseline; 1.1729x over previous
import functools

import jax
import jax.numpy as jnp
from jax import lax
from jax.experimental import pallas as pl
from jax.experimental.pallas import tpu as pltpu

N_DEV = 4
KV_CHUNK = 256
NBUF = 3


def _flash_partial_body(
    q_hbm, k_hbm, v_hbm, o_ref, m_ref, l_ref, qb, kb, vb, qsem, sems, *, scale
):
    B, Sq, H, D = q_hbm.shape
    Skv = k_hbm.shape[1]
    C = KV_CHUNK
    NC = Skv // C

    pltpu.make_async_copy(q_hbm, qb, qsem).start()

    def start(i):
        b, c = divmod(i, NC)
        slot = i % NBUF
        for h in range(H):
            pltpu.make_async_copy(
                k_hbm.at[b, pl.ds(c * C, C), h, :], kb.at[slot, h], sems.at[slot, 0]
            ).start()
            pltpu.make_async_copy(
                v_hbm.at[b, pl.ds(c * C, C), h, :], vb.at[slot, h], sems.at[slot, 1]
            ).start()

    NIT = B * NC
    for s in range(min(NBUF, NIT)):
        start(s)

    pltpu.make_async_copy(q_hbm, qb, qsem).wait()

    for b in range(B):
        qs = (jnp.transpose(qb[b], (1, 0, 2)) * scale).astype(jnp.bfloat16)
        acc = None
        m = None
        l = None
        for c in range(NC):
            i = b * NC + c
            slot = i % NBUF
            for h in range(H):
                pltpu.make_async_copy(
                    k_hbm.at[0, pl.ds(0, C), 0, :], kb.at[0, 0], sems.at[slot, 0]
                ).wait()
                pltpu.make_async_copy(
                    v_hbm.at[0, pl.ds(0, C), 0, :], vb.at[0, 0], sems.at[slot, 1]
                ).wait()
            k = kb[slot].astype(jnp.bfloat16)
            v = vb[slot].astype(jnp.bfloat16)
            nxt = i + NBUF
            if nxt < NIT:
                start(nxt)
            s = lax.dot_general(
                qs, k, (((2,), (2,)), ((0,), (0,))),
                preferred_element_type=jnp.float32,
            )
            m_c = jnp.max(s, axis=-1)
            if acc is None:
                m = m_c
                p = jnp.exp(s - m[:, :, None])
                l = jnp.sum(p, axis=-1)
                acc = lax.dot_general(
                    p.astype(jnp.bfloat16), v, (((2,), (1,)), ((0,), (0,))),
                    preferred_element_type=jnp.float32,
                )
            else:
                m_new = jnp.maximum(m, m_c)
                alpha = jnp.exp(m - m_new)
                p = jnp.exp(s - m_new[:, :, None])
                pv = lax.dot_general(
                    p.astype(jnp.bfloat16), v, (((2,), (1,)), ((0,), (0,))),
                    preferred_element_type=jnp.float32,
                )
                acc = acc * alpha[:, :, None] + pv
                l = l * alpha + jnp.sum(p, axis=-1)
                m = m_new
        o_ref[b] = jnp.transpose(acc, (1, 0, 2)).astype(jnp.bfloat16)
        m_ref[b] = m.T
        l_ref[b] = l.T


def _flash_partial(Q, K, V):
    B, Sq, H, D = Q.shape
    Skv = K.shape[1]
    scale = D**-0.5
    body = functools.partial(_flash_partial_body, scale=scale)
    return pl.pallas_call(
        body,
        in_specs=[
            pl.BlockSpec(memory_space=pl.ANY),
            pl.BlockSpec(memory_space=pl.ANY),
            pl.BlockSpec(memory_space=pl.ANY),
        ],
        out_specs=[
            pl.BlockSpec(memory_space=pltpu.VMEM),
            pl.BlockSpec(memory_space=pltpu.VMEM),
            pl.BlockSpec(memory_space=pltpu.VMEM),
        ],
        out_shape=[
            jax.ShapeDtypeStruct((B, Sq, H, D), jnp.bfloat16),
            jax.ShapeDtypeStruct((B, Sq, H), jnp.float32),
            jax.ShapeDtypeStruct((B, Sq, H), jnp.float32),
        ],
        scratch_shapes=[
            pltpu.VMEM((B, Sq, H, D), jnp.float32),
            pltpu.VMEM((NBUF, H, KV_CHUNK, D), jnp.float32),
            pltpu.VMEM((NBUF, H, KV_CHUNK, D), jnp.float32),
            pltpu.SemaphoreType.DMA,
            pltpu.SemaphoreType.DMA((NBUF, 2)),
        ],
    )(Q, K, V)


def _fused_body(
    q_hbm,
    k_hbm,
    v_hbm,
    out_ref,
    qb,
    kb,
    vb,
    sO,
    sML,
    s1O,
    s1ML,
    rO,
    rML,
    qsem,
    sems,
    ssem,
    rsem,
    *,
    scale,
):
    B, Sq, H, D = q_hbm.shape
    Skv = k_hbm.shape[1]
    C = KV_CHUNK
    NC = Skv // C

    my = lax.axis_index("i")
    p1 = jnp.bitwise_xor(my, 1)
    p2 = 3 - my

    barrier = pltpu.get_barrier_semaphore()
    for nbr in (p1, p2):
        pl.semaphore_signal(
            barrier, inc=1, device_id=(nbr,), device_id_type=pl.DeviceIdType.MESH
        )
    pl.semaphore_wait(barrier, 2)

    def rdma(r, b, partner, j):
        src = (sO, s1O)[r] if j == 0 else (sML, s1ML)[r]
        dst = rO if j == 0 else rML
        return pltpu.make_async_remote_copy(
            src_ref=src.at[b],
            dst_ref=dst.at[r, b],
            send_sem=ssem.at[r, b, j],
            recv_sem=rsem.at[r, b, j],
            device_id=(partner,),
            device_id_type=pl.DeviceIdType.MESH,
        )

    pltpu.make_async_copy(q_hbm, qb, qsem).start()

    def start(i):
        b, c = divmod(i, NC)
        slot = i % NBUF
        for h in range(H):
            pltpu.make_async_copy(
                k_hbm.at[b, pl.ds(c * C, C), h, :], kb.at[slot, h], sems.at[slot, 0]
            ).start()
            pltpu.make_async_copy(
                v_hbm.at[b, pl.ds(c * C, C), h, :], vb.at[slot, h], sems.at[slot, 1]
            ).start()

    NIT = B * NC
    for s in range(min(NBUF, NIT)):
        start(s)

    pltpu.make_async_copy(q_hbm, qb, qsem).wait()

    def combine(o_a_f32, m_a, l_a, r, b):
        m_b = rML[r, b, 0]
        l_b = rML[r, b, 1]
        m_new = jnp.maximum(m_a, m_b)
        w_a = jnp.exp(m_a - m_new)
        w_b = jnp.exp(m_b - m_new)
        l_new = l_a * w_a + l_b * w_b
        o_new = (
            o_a_f32 * w_a[..., None]
            + rO[r, b].astype(jnp.float32) * w_b[..., None]
        )
        return o_new, m_new, l_new

    def compute_b(b):
        qs = jnp.transpose(qb[b], (1, 0, 2)) * scale
        acc = None
        m = None
        l = None
        for c in range(NC):
            i = b * NC + c
            slot = i % NBUF
            for h in range(H):
                pltpu.make_async_copy(
                    k_hbm.at[0, pl.ds(0, C), 0, :], kb.at[0, 0], sems.at[slot, 0]
                ).wait()
                pltpu.make_async_copy(
                    v_hbm.at[0, pl.ds(0, C), 0, :], vb.at[0, 0], sems.at[slot, 1]
                ).wait()
            k = kb[slot]
            v = vb[slot]
            nxt = i + NBUF
            if nxt < NIT:
                start(nxt)
            s = lax.dot_general(
                qs, k, (((2,), (2,)), ((0,), (0,))),
                preferred_element_type=jnp.float32,
            )
            m_c = jnp.max(s, axis=-1)
            if acc is None:
                m = m_c
                p = jnp.exp(s - m[:, :, None])
                l = jnp.sum(p, axis=-1)
                acc = lax.dot_general(
                    p, v, (((2,), (1,)), ((0,), (0,))),
                    preferred_element_type=jnp.float32,
                )
            else:
                m_new = jnp.maximum(m, m_c)
                alpha = jnp.exp(m - m_new)
                p = jnp.exp(s - m_new[:, :, None])
                pv = lax.dot_general(
                    p, v, (((2,), (1,)), ((0,), (0,))),
                    preferred_element_type=jnp.float32,
                )
                acc = acc * alpha[:, :, None] + pv
                l = l * alpha + jnp.sum(p, axis=-1)
                m = m_new
        sO[b] = jnp.transpose(acc, (1, 0, 2)).astype(jnp.bfloat16)
        sML[b, 0] = m.T
        sML[b, 1] = l.T
        rdma(0, b, p1, 0).start()
        rdma(0, b, p1, 1).start()

    def phase_a(b):
        for j in (0, 1):
            rdma(0, b, p1, j).wait_recv()
        o1, m1, l1 = combine(sO[b].astype(jnp.float32), sML[b, 0], sML[b, 1], 0, b)
        s1O[b] = o1.astype(jnp.bfloat16)
        s1ML[b, 0] = m1
        s1ML[b, 1] = l1
        rdma(1, b, p2, 0).start()
        rdma(1, b, p2, 1).start()

    def phase_b(b):
        for j in (0, 1):
            rdma(1, b, p2, j).wait_recv()
        o2, _, l2 = combine(s1O[b].astype(jnp.float32), s1ML[b, 0], s1ML[b, 1], 1, b)
        out_ref[b] = o2 / l2[..., None]

    for b in range(B):
        compute_b(b)
        if b >= 2:
            phase_a(b - 2)
        if b >= 4:
            phase_b(b - 4)
    for b in (B - 2, B - 1):
        phase_a(b)
    for b in range(B - 4, B):
        phase_b(b)

    for b in range(B):
        for j in (0, 1):
            rdma(0, b, p1, j).wait_send()
            rdma(1, b, p2, j).wait_send()


def _fused(Q, K, V):
    B, Sq, H, D = Q.shape
    body = functools.partial(_fused_body, scale=D**-0.5)
    return pl.pallas_call(
        body,
        in_specs=[pl.BlockSpec(memory_space=pl.ANY)] * 3,
        out_specs=pl.BlockSpec(memory_space=pltpu.VMEM),
        out_shape=jax.ShapeDtypeStruct((B, Sq, H, D), jnp.float32),
        scratch_shapes=[
            pltpu.VMEM((B, Sq, H, D), jnp.float32),
            pltpu.VMEM((NBUF, H, KV_CHUNK, D), jnp.float32),
            pltpu.VMEM((NBUF, H, KV_CHUNK, D), jnp.float32),
            pltpu.VMEM((B, Sq, H, D), jnp.bfloat16),
            pltpu.VMEM((B, 2, Sq, H), jnp.float32),
            pltpu.VMEM((B, Sq, H, D), jnp.bfloat16),
            pltpu.VMEM((B, 2, Sq, H), jnp.float32),
            pltpu.VMEM((2, B, Sq, H, D), jnp.bfloat16),
            pltpu.VMEM((2, B, 2, Sq, H), jnp.float32),
            pltpu.SemaphoreType.DMA,
            pltpu.SemaphoreType.DMA((NBUF, 2)),
            pltpu.SemaphoreType.DMA((2, B, 2)),
            pltpu.SemaphoreType.DMA((2, B, 2)),
        ],
        compiler_params=pltpu.CompilerParams(collective_id=0),
    )(Q, K, V)


def _allreduce_body(
    o_ref, m_ref, l_ref, out_ref, obuf, mbuf, lbuf, co, cm, cl, send_sems, recv_sems
):
    my = lax.axis_index("i")
    p1 = jnp.bitwise_xor(my, 1)
    p2 = 3 - my

    barrier = pltpu.get_barrier_semaphore()
    for nbr in (p1, p2):
        pl.semaphore_signal(
            barrier, inc=1, device_id=(nbr,), device_id_type=pl.DeviceIdType.MESH
        )
    pl.semaphore_wait(barrier, 2)

    def exchange(r, partner, src_o, src_m, src_l):
        copies = []
        for j, (src, dst) in enumerate(
            ((src_o, obuf), (src_m, mbuf), (src_l, lbuf))
        ):
            rdma = pltpu.make_async_remote_copy(
                src_ref=src,
                dst_ref=dst.at[r],
                send_sem=send_sems.at[r, j],
                recv_sem=recv_sems.at[r, j],
                device_id=(partner,),
                device_id_type=pl.DeviceIdType.MESH,
            )
            rdma.start()
            copies.append(rdma)
        for rdma in copies:
            rdma.wait()

    def combine(r, src_o, src_m, src_l):
        m_a = src_m[...]
        m_b = mbuf[r]
        m_new = jnp.maximum(m_a, m_b)
        w_a = jnp.exp(m_a - m_new)
        w_b = jnp.exp(m_b - m_new)
        l_new = src_l[...] * w_a + lbuf[r] * w_b
        o_new = (
            src_o[...].astype(jnp.float32) * w_a[..., None]
            + obuf[r].astype(jnp.float32) * w_b[..., None]
        )
        return o_new, m_new, l_new

    exchange(0, p1, o_ref, m_ref, l_ref)
    o1, m1, l1 = combine(0, o_ref, m_ref, l_ref)
    co[...] = o1.astype(jnp.bfloat16)
    cm[...] = m1
    cl[...] = l1
    exchange(1, p2, co, cm, cl)
    o2, _, l2 = combine(1, co, cm, cl)
    out_ref[...] = o2 / l2[..., None]


def _combine_allreduce(o, m, l):
    B, Sq, H, D = o.shape
    return pl.pallas_call(
        _allreduce_body,
        out_shape=jax.ShapeDtypeStruct((B, Sq, H, D), jnp.float32),
        in_specs=[pl.BlockSpec(memory_space=pltpu.VMEM)] * 3,
        out_specs=pl.BlockSpec(memory_space=pltpu.VMEM),
        scratch_shapes=[
            pltpu.VMEM((2, B, Sq, H, D), jnp.bfloat16),
            pltpu.VMEM((2, B, Sq, H), jnp.float32),
            pltpu.VMEM((2, B, Sq, H), jnp.float32),
            pltpu.VMEM((B, Sq, H, D), jnp.bfloat16),
            pltpu.VMEM((B, Sq, H), jnp.float32),
            pltpu.VMEM((B, Sq, H), jnp.float32),
            pltpu.SemaphoreType.DMA((2, 3)),
            pltpu.SemaphoreType.DMA((2, 3)),
        ],
        compiler_params=pltpu.CompilerParams(collective_id=0),
    )(o, m, l)


def kernel(Q, K, V):
    return _fused(Q, K, V)


def kernel_two_call(Q, K, V):
    o, m, l = _flash_partial(Q, K, V)
    return _combine_allreduce(o, m, l)
